# baseline (speedup 1.0000x reference)
import functools

import jax
import jax.numpy as jnp
from jax.experimental import pallas as pl

N_E = 8192
E_DIM = 32
BETA = 0.25
TILE = 256


def _vq_tile(zf_ref, sz_ref, et_ref, e_ref, zq_ref, idx_ref, loss_ref):
    zf = zf_ref[...]
    sz = sz_ref[...]
    et = et_ref[...]
    e = e_ref[...]
    se = jnp.sum(et * et, axis=0, keepdims=True)
    zfb = zf.astype(jnp.bfloat16).astype(jnp.float32)
    mm = jax.lax.dot_general(
        zfb, et, (((1,), (0,)), ((), ())),
        preferred_element_type=jnp.float32)
    d = (sz + se) - 2.0 * mm
    iota = jax.lax.broadcasted_iota(jnp.int32, d.shape, 1)
    CHUNK = N_E // 2
    acc_v = None
    acc_i = None
    for c in range(2):
        seg = d[:, c * CHUNK:(c + 1) * CHUNK]
        m_c = jnp.min(seg, axis=1)
        ii = iota[:, c * CHUNK:(c + 1) * CHUNK]
        i_c = jnp.min(jnp.where(seg == m_c[:, None], ii, N_E), axis=1)
        if acc_v is None:
            acc_v, acc_i = m_c, i_c
        else:
            upd = m_c < acc_v
            acc_v = jnp.where(upd, m_c, acc_v)
            acc_i = jnp.where(upd, i_c, acc_i)
        acc_v = acc_v.astype(jnp.bfloat16).astype(jnp.float32)
    idx = acc_i
    idx_ref[...] = idx[:, None]
    oh = (iota == idx[:, None]).astype(jnp.float32)
    zq = jax.lax.dot_general(
        oh, e, (((1,), (0,)), ((), ())),
        preferred_element_type=jnp.float32)
    diff = zq - zf
    zq_ref[...] = zf + diff
    loss_ref[...] = jnp.sum(diff * diff).reshape(1, 1, 1)


@functools.partial(jax.jit, static_argnames=())
def kernel(z, embedding_weight):
    B, C, H, W = z.shape
    n = B * H * W
    grid = n // TILE
    zp = jnp.transpose(z, (0, 2, 3, 1))
    zf = zp.reshape(n, E_DIM)
    sz = jnp.sum(zp ** 2, axis=3).reshape(n, 1)
    et = embedding_weight.T

    zq_flat, idx_col, loss_parts = pl.pallas_call(
        _vq_tile,
        grid=(grid,),
        in_specs=[
            pl.BlockSpec((TILE, E_DIM), lambda i: (i, 0)),
            pl.BlockSpec((TILE, 1), lambda i: (i, 0)),
            pl.BlockSpec((E_DIM, N_E), lambda i: (0, 0)),
            pl.BlockSpec((N_E, E_DIM), lambda i: (0, 0)),
        ],
        out_specs=[
            pl.BlockSpec((TILE, E_DIM), lambda i: (i, 0)),
            pl.BlockSpec((TILE, 1), lambda i: (i, 0)),
            pl.BlockSpec((1, 1, 1), lambda i: (i, 0, 0)),
        ],
        out_shape=[
            jax.ShapeDtypeStruct((n, E_DIM), jnp.float32),
            jax.ShapeDtypeStruct((n, 1), jnp.int32),
            jax.ShapeDtypeStruct((grid, 1, 1), jnp.float32),
        ],
    )(zf, sz, et, embedding_weight)

    mse = jnp.sum(loss_parts) / (n * E_DIM)
    loss = mse + BETA * mse
    z_q_out = jnp.transpose(zq_flat.reshape(B, H, W, C), (0, 3, 1, 2))
    idx_out = idx_col.reshape(B, H, W)
    return (z_q_out, loss, idx_out)


# speedup vs baseline: 1.6864x; 1.6864x over previous
import functools

import jax
import jax.numpy as jnp
from jax import lax
from jax.experimental import pallas as pl
from jax.experimental.pallas import tpu as pltpu, tpu_sc as plsc

N_E = 8192
E_DIM = 32
BETA = 0.25
TILE = 256


def _vq_tile(zf_ref, sz_ref, et_ref, idx_ref, loss_ref):
    zf = zf_ref[...]
    sz = sz_ref[...]
    et = et_ref[...]
    se = jnp.sum(et * et, axis=0, keepdims=True)
    zfb = zf.astype(jnp.bfloat16).astype(jnp.float32)
    mm = jax.lax.dot_general(
        zfb, et, (((1,), (0,)), ((), ())),
        preferred_element_type=jnp.float32)
    d = (sz + se) - 2.0 * mm
    iota = jax.lax.broadcasted_iota(jnp.int32, d.shape, 1)
    CHUNK = N_E // 2
    acc_v = None
    acc_x = None
    acc_i = None
    for c in range(2):
        seg = d[:, c * CHUNK:(c + 1) * CHUNK]
        m_c = jnp.min(seg, axis=1)
        ii = iota[:, c * CHUNK:(c + 1) * CHUNK]
        i_c = jnp.min(jnp.where(seg == m_c[:, None], ii, N_E), axis=1)
        if acc_v is None:
            acc_v, acc_x, acc_i = m_c, m_c, i_c
        else:
            upd = m_c < acc_v
            acc_v = jnp.where(upd, m_c, acc_v)
            acc_x = jnp.where(upd, m_c, acc_x)
            acc_i = jnp.where(upd, i_c, acc_i)
        acc_v = acc_v.astype(jnp.bfloat16).astype(jnp.float32)
    idx_ref[...] = acc_i[:, None]
    loss_ref[...] = jnp.sum(acc_x).reshape(1, 1, 1)


def _sc_gather(table_hbm, idx_hbm, out_hbm, idx_v, rows_v, sem):
    info = plsc.get_sparse_core_info()
    nw = info.num_cores * info.num_subcores
    b_per_w = idx_hbm.shape[0] // nw
    wid = lax.axis_index("s") * info.num_cores + lax.axis_index("c")
    base = wid * b_per_w
    pltpu.sync_copy(idx_hbm.at[pl.ds(base, b_per_w)], idx_v)
    pltpu.async_copy(table_hbm.at[idx_v], rows_v, sem).wait()
    pltpu.sync_copy(rows_v, out_hbm.at[pl.ds(base, b_per_w)])


@functools.partial(jax.jit, static_argnames=())
def kernel(z, embedding_weight):
    B, C, H, W = z.shape
    n = B * H * W
    grid = n // TILE
    zp = jnp.transpose(z, (0, 2, 3, 1))
    zf = zp.reshape(n, E_DIM)
    sz = jnp.sum(zp ** 2, axis=3).reshape(n, 1)
    et = embedding_weight.T

    idx_col, loss_parts = pl.pallas_call(
        _vq_tile,
        grid=(grid,),
        in_specs=[
            pl.BlockSpec((TILE, E_DIM), lambda i: (i, 0)),
            pl.BlockSpec((TILE, 1), lambda i: (i, 0)),
            pl.BlockSpec((E_DIM, N_E), lambda i: (0, 0)),
        ],
        out_specs=[
            pl.BlockSpec((TILE, 1), lambda i: (i, 0)),
            pl.BlockSpec((1, 1, 1), lambda i: (i, 0, 0)),
        ],
        out_shape=[
            jax.ShapeDtypeStruct((n, 1), jnp.int32),
            jax.ShapeDtypeStruct((grid, 1, 1), jnp.float32),
        ],
    )(zf, sz, et)

    idx_flat = idx_col.reshape(n)

    table = jnp.pad(embedding_weight, ((0, 0), (0, 128 - E_DIM)))

    info = plsc.get_sparse_core_info()
    nw = info.num_cores * info.num_subcores
    b_per_w = n // nw
    mesh = plsc.VectorSubcoreMesh(core_axis_name="c", subcore_axis_name="s")
    zq_pad = pl.kernel(
        _sc_gather,
        mesh=mesh,
        out_type=jax.ShapeDtypeStruct((n, 128), jnp.float32),
        scratch_types=[
            pltpu.VMEM((b_per_w,), jnp.int32),
            pltpu.VMEM((b_per_w, 128), jnp.float32),
            pltpu.SemaphoreType.DMA,
        ],
    )(table, idx_flat)
    zq_flat = zq_pad[:, :E_DIM]

    mse = jnp.sum(loss_parts) / (n * E_DIM)
    loss = mse + BETA * mse
    z_q_out = jnp.transpose(zq_flat.reshape(B, H, W, C), (0, 3, 1, 2))
    idx_out = idx_col.reshape(B, H, W)
    return (z_q_out, loss, idx_out)


# speedup vs baseline: 1.6970x; 1.0063x over previous
import functools

import jax
import jax.numpy as jnp
from jax import lax
from jax.experimental import pallas as pl
from jax.experimental.pallas import tpu as pltpu, tpu_sc as plsc

N_E = 8192
E_DIM = 32
BETA = 0.25
TILE = 256


def _vq_tile(zf_ref, sz_ref, et_ref, idx_ref, loss_ref, se_scr):
    zf = zf_ref[...]
    sz = sz_ref[...]
    et = et_ref[...]

    @pl.when(pl.program_id(0) == 0)
    def _():
        se_scr[...] = jnp.sum(et * et, axis=0, keepdims=True)

    se = se_scr[...]
    zfb = zf.astype(jnp.bfloat16).astype(jnp.float32)
    mm = jax.lax.dot_general(
        zfb, et, (((1,), (0,)), ((), ())),
        preferred_element_type=jnp.float32)
    d = (sz + se) - 2.0 * mm
    iota = jax.lax.broadcasted_iota(jnp.int32, d.shape, 1)
    CHUNK = N_E // 2
    acc_v = None
    acc_x = None
    acc_i = None
    for c in range(2):
        seg = d[:, c * CHUNK:(c + 1) * CHUNK]
        m_c = jnp.min(seg, axis=1)
        ii = iota[:, c * CHUNK:(c + 1) * CHUNK]
        i_c = jnp.min(jnp.where(seg == m_c[:, None], ii, N_E), axis=1)
        if acc_v is None:
            acc_v, acc_x, acc_i = m_c, m_c, i_c
        else:
            upd = m_c < acc_v
            acc_v = jnp.where(upd, m_c, acc_v)
            acc_x = jnp.where(upd, m_c, acc_x)
            acc_i = jnp.where(upd, i_c, acc_i)
        acc_v = acc_v.astype(jnp.bfloat16).astype(jnp.float32)
    idx_ref[...] = acc_i[:, None]
    loss_ref[...] = jnp.sum(acc_x).reshape(1, 1, 1)


def _sc_gather(table_hbm, idx_hbm, out_hbm, idx_v, rows_v, sem):
    info = plsc.get_sparse_core_info()
    nw = info.num_cores * info.num_subcores
    b_per_w = idx_hbm.shape[0] // nw
    wid = lax.axis_index("s") * info.num_cores + lax.axis_index("c")
    base = wid * b_per_w
    pltpu.sync_copy(idx_hbm.at[pl.ds(base, b_per_w)], idx_v)
    pltpu.async_copy(table_hbm.at[idx_v], rows_v, sem).wait()
    pltpu.sync_copy(rows_v, out_hbm.at[pl.ds(base, b_per_w)])


@functools.partial(jax.jit, static_argnames=())
def kernel(z, embedding_weight):
    B, C, H, W = z.shape
    n = B * H * W
    grid = n // TILE
    zp = jnp.transpose(z, (0, 2, 3, 1))
    zf = zp.reshape(n, E_DIM)
    sz = jnp.sum(zp ** 2, axis=3).reshape(n, 1)
    et = embedding_weight.T

    idx_col, loss_parts = pl.pallas_call(
        _vq_tile,
        grid=(grid,),
        in_specs=[
            pl.BlockSpec((TILE, E_DIM), lambda i: (i, 0)),
            pl.BlockSpec((TILE, 1), lambda i: (i, 0)),
            pl.BlockSpec((E_DIM, N_E), lambda i: (0, 0)),
        ],
        out_specs=[
            pl.BlockSpec((TILE, 1), lambda i: (i, 0)),
            pl.BlockSpec((1, 1, 1), lambda i: (i, 0, 0)),
        ],
        out_shape=[
            jax.ShapeDtypeStruct((n, 1), jnp.int32),
            jax.ShapeDtypeStruct((grid, 1, 1), jnp.float32),
        ],
        scratch_shapes=[pltpu.VMEM((1, N_E), jnp.float32)],
    )(zf, sz, et)

    idx_flat = idx_col.reshape(n)

    table = jnp.pad(embedding_weight, ((0, 0), (0, 128 - E_DIM)))

    info = plsc.get_sparse_core_info()
    nw = info.num_cores * info.num_subcores
    b_per_w = n // nw
    mesh = plsc.VectorSubcoreMesh(core_axis_name="c", subcore_axis_name="s")
    zq_pad = pl.kernel(
        _sc_gather,
        mesh=mesh,
        out_type=jax.ShapeDtypeStruct((n, 128), jnp.float32),
        scratch_types=[
            pltpu.VMEM((b_per_w,), jnp.int32),
            pltpu.VMEM((b_per_w, 128), jnp.float32),
            pltpu.SemaphoreType.DMA,
        ],
    )(table, idx_flat)
    zq_flat = zq_pad[:, :E_DIM]

    mse = jnp.sum(loss_parts) / (n * E_DIM)
    loss = mse + BETA * mse
    z_q_out = jnp.transpose(zq_flat.reshape(B, H, W, C), (0, 3, 1, 2))
    idx_out = idx_col.reshape(B, H, W)
    return (z_q_out, loss, idx_out)


# speedup vs baseline: 1.7775x; 1.0475x over previous
import functools

import jax
import jax.numpy as jnp
from jax import lax
from jax.experimental import pallas as pl
from jax.experimental.pallas import tpu as pltpu, tpu_sc as plsc

N_E = 8192
E_DIM = 32
BETA = 0.25
TILE = 512


def _vq_tile(zf_ref, sz_ref, et_ref, idx_ref, loss_ref, se_scr):
    zf = zf_ref[...]
    sz = sz_ref[...]
    et = et_ref[...]

    @pl.when(pl.program_id(0) == 0)
    def _():
        se_scr[...] = jnp.sum(et * et, axis=0, keepdims=True)

    se = se_scr[...]
    zfb = zf.astype(jnp.bfloat16).astype(jnp.float32)
    mm = jax.lax.dot_general(
        zfb, et, (((1,), (0,)), ((), ())),
        preferred_element_type=jnp.float32)
    d = (sz + se) - 2.0 * mm
    iota = jax.lax.broadcasted_iota(jnp.int32, d.shape, 1)
    CHUNK = N_E // 2
    acc_v = None
    acc_x = None
    acc_i = None
    for c in range(2):
        seg = d[:, c * CHUNK:(c + 1) * CHUNK]
        m_c = jnp.min(seg, axis=1)
        ii = iota[:, c * CHUNK:(c + 1) * CHUNK]
        i_c = jnp.min(jnp.where(seg == m_c[:, None], ii, N_E), axis=1)
        if acc_v is None:
            acc_v, acc_x, acc_i = m_c, m_c, i_c
        else:
            upd = m_c < acc_v
            acc_v = jnp.where(upd, m_c, acc_v)
            acc_x = jnp.where(upd, m_c, acc_x)
            acc_i = jnp.where(upd, i_c, acc_i)
        acc_v = acc_v.astype(jnp.bfloat16).astype(jnp.float32)
    idx_ref[...] = acc_i[:, None]
    loss_ref[...] = jnp.sum(acc_x).reshape(1, 1, 1)


def _sc_gather(table_hbm, idx_hbm, out_hbm, idx_v, rows_v, sem):
    info = plsc.get_sparse_core_info()
    nw = info.num_cores * info.num_subcores
    b_per_w = idx_hbm.shape[0] // nw
    wid = lax.axis_index("s") * info.num_cores + lax.axis_index("c")
    base = wid * b_per_w
    pltpu.sync_copy(idx_hbm.at[pl.ds(base, b_per_w)], idx_v)
    pltpu.async_copy(table_hbm.at[idx_v], rows_v, sem).wait()
    pltpu.sync_copy(rows_v, out_hbm.at[pl.ds(base, b_per_w)])


@functools.partial(jax.jit, static_argnames=())
def kernel(z, embedding_weight):
    B, C, H, W = z.shape
    n = B * H * W
    grid = n // TILE
    zp = jnp.transpose(z, (0, 2, 3, 1))
    zf = zp.reshape(n, E_DIM)
    sz = jnp.sum(zp ** 2, axis=3).reshape(n, 1)
    et = embedding_weight.T

    idx_col, loss_parts = pl.pallas_call(
        _vq_tile,
        grid=(grid,),
        in_specs=[
            pl.BlockSpec((TILE, E_DIM), lambda i: (i, 0)),
            pl.BlockSpec((TILE, 1), lambda i: (i, 0)),
            pl.BlockSpec((E_DIM, N_E), lambda i: (0, 0)),
        ],
        out_specs=[
            pl.BlockSpec((TILE, 1), lambda i: (i, 0)),
            pl.BlockSpec((1, 1, 1), lambda i: (i, 0, 0)),
        ],
        out_shape=[
            jax.ShapeDtypeStruct((n, 1), jnp.int32),
            jax.ShapeDtypeStruct((grid, 1, 1), jnp.float32),
        ],
        scratch_shapes=[pltpu.VMEM((1, N_E), jnp.float32)],
    )(zf, sz, et)

    idx_flat = idx_col.reshape(n)

    table = jnp.pad(embedding_weight, ((0, 0), (0, 128 - E_DIM)))

    info = plsc.get_sparse_core_info()
    nw = info.num_cores * info.num_subcores
    b_per_w = n // nw
    mesh = plsc.VectorSubcoreMesh(core_axis_name="c", subcore_axis_name="s")
    zq_pad = pl.kernel(
        _sc_gather,
        mesh=mesh,
        out_type=jax.ShapeDtypeStruct((n, 128), jnp.float32),
        scratch_types=[
            pltpu.VMEM((b_per_w,), jnp.int32),
            pltpu.VMEM((b_per_w, 128), jnp.float32),
            pltpu.SemaphoreType.DMA,
        ],
    )(table, idx_flat)
    zq_flat = zq_pad[:, :E_DIM]

    mse = jnp.sum(loss_parts) / (n * E_DIM)
    loss = mse + BETA * mse
    z_q_out = jnp.transpose(zq_flat.reshape(B, H, W, C), (0, 3, 1, 2))
    idx_out = idx_col.reshape(B, H, W)
    return (z_q_out, loss, idx_out)


# speedup vs baseline: 2.2099x; 1.2433x over previous
import functools

import jax
import jax.numpy as jnp
from jax import lax
from jax.experimental import pallas as pl
from jax.experimental.pallas import tpu as pltpu, tpu_sc as plsc

N_E = 8192
E_DIM = 32
BETA = 0.25
TILE = 512


def _vq_tile(zf_ref, sz_ref, et_ref, idx_ref, loss_ref, se_scr):
    zf = zf_ref[...]
    sz = sz_ref[...]
    et = et_ref[...]

    @pl.when(pl.program_id(0) == 0)
    def _():
        se_scr[...] = jnp.sum(et * et, axis=0, keepdims=True)

    se = se_scr[...]
    zfb = zf.astype(jnp.bfloat16).astype(jnp.float32)
    mm2 = jax.lax.dot_general(
        zfb + zfb, et, (((1,), (0,)), ((), ())),
        preferred_element_type=jnp.float32)
    CHUNK = N_E // 2
    STRIP = 128
    lane = jax.lax.broadcasted_iota(jnp.int32, (zf.shape[0], STRIP), 1)
    acc_v = None
    acc_x = None
    acc_i = None
    for c in range(2):
        mv = None
        gv = None
        for g in range(CHUNK // STRIP):
            j0 = c * CHUNK + g * STRIP
            d_s = (sz + se[:, j0:j0 + STRIP]) - mm2[:, j0:j0 + STRIP]
            if mv is None:
                mv, gv = d_s, jnp.zeros_like(lane)
            else:
                upd = d_s < mv
                mv = jnp.where(upd, d_s, mv)
                gv = jnp.where(upd, g, gv)
        jv = gv * STRIP + lane + c * CHUNK
        m_c = jnp.min(mv, axis=1)
        i_c = jnp.min(jnp.where(mv == m_c[:, None], jv, N_E), axis=1)
        if acc_v is None:
            acc_v, acc_x, acc_i = m_c, m_c, i_c
        else:
            upd = m_c < acc_v
            acc_v = jnp.where(upd, m_c, acc_v)
            acc_x = jnp.where(upd, m_c, acc_x)
            acc_i = jnp.where(upd, i_c, acc_i)
        acc_v = acc_v.astype(jnp.bfloat16).astype(jnp.float32)
    idx_ref[...] = acc_i[:, None]
    loss_ref[...] = jnp.sum(acc_x).reshape(1, 1, 1)


def _sc_gather(table_hbm, idx_hbm, out_hbm, idx_v, rows_v, sem):
    info = plsc.get_sparse_core_info()
    nw = info.num_cores * info.num_subcores
    b_per_w = idx_hbm.shape[0] // nw
    wid = lax.axis_index("s") * info.num_cores + lax.axis_index("c")
    base = wid * b_per_w
    pltpu.sync_copy(idx_hbm.at[pl.ds(base, b_per_w)], idx_v)
    pltpu.async_copy(table_hbm.at[idx_v], rows_v, sem).wait()
    pltpu.sync_copy(rows_v, out_hbm.at[pl.ds(base, b_per_w)])


@functools.partial(jax.jit, static_argnames=())
def kernel(z, embedding_weight):
    B, C, H, W = z.shape
    n = B * H * W
    grid = n // TILE
    zp = jnp.transpose(z, (0, 2, 3, 1))
    zf = zp.reshape(n, E_DIM)
    sz = jnp.sum(zp ** 2, axis=3).reshape(n, 1)
    et = embedding_weight.T

    idx_col, loss_parts = pl.pallas_call(
        _vq_tile,
        grid=(grid,),
        in_specs=[
            pl.BlockSpec((TILE, E_DIM), lambda i: (i, 0)),
            pl.BlockSpec((TILE, 1), lambda i: (i, 0)),
            pl.BlockSpec((E_DIM, N_E), lambda i: (0, 0)),
        ],
        out_specs=[
            pl.BlockSpec((TILE, 1), lambda i: (i, 0)),
            pl.BlockSpec((1, 1, 1), lambda i: (i, 0, 0)),
        ],
        out_shape=[
            jax.ShapeDtypeStruct((n, 1), jnp.int32),
            jax.ShapeDtypeStruct((grid, 1, 1), jnp.float32),
        ],
        scratch_shapes=[pltpu.VMEM((1, N_E), jnp.float32)],
    )(zf, sz, et)

    idx_flat = idx_col.reshape(n)

    table = jnp.pad(embedding_weight, ((0, 0), (0, 128 - E_DIM)))

    info = plsc.get_sparse_core_info()
    nw = info.num_cores * info.num_subcores
    b_per_w = n // nw
    mesh = plsc.VectorSubcoreMesh(core_axis_name="c", subcore_axis_name="s")
    zq_pad = pl.kernel(
        _sc_gather,
        mesh=mesh,
        out_type=jax.ShapeDtypeStruct((n, 128), jnp.float32),
        scratch_types=[
            pltpu.VMEM((b_per_w,), jnp.int32),
            pltpu.VMEM((b_per_w, 128), jnp.float32),
            pltpu.SemaphoreType.DMA,
        ],
    )(table, idx_flat)
    zq_flat = zq_pad[:, :E_DIM]

    mse = jnp.sum(loss_parts) / (n * E_DIM)
    loss = mse + BETA * mse
    z_q_out = jnp.transpose(zq_flat.reshape(B, H, W, C), (0, 3, 1, 2))
    idx_out = idx_col.reshape(B, H, W)
    return (z_q_out, loss, idx_out)


# speedup vs baseline: 2.2962x; 1.0391x over previous
import functools

import jax
import jax.numpy as jnp
from jax import lax
from jax.experimental import pallas as pl
from jax.experimental.pallas import tpu as pltpu, tpu_sc as plsc

N_E = 8192
E_DIM = 32
BETA = 0.25
TILE = 1024


def _vq_tile(zf_ref, sz_ref, et_ref, idx_ref, loss_ref, se_scr):
    zf = zf_ref[...]
    sz = sz_ref[...]
    et = et_ref[...]

    @pl.when(pl.program_id(0) == 0)
    def _():
        se_scr[...] = jnp.sum(et * et, axis=0, keepdims=True)

    se = se_scr[...]
    zfb = zf.astype(jnp.bfloat16).astype(jnp.float32)
    mm2 = jax.lax.dot_general(
        zfb + zfb, et, (((1,), (0,)), ((), ())),
        preferred_element_type=jnp.float32)
    CHUNK = N_E // 2
    STRIP = 128
    lane = jax.lax.broadcasted_iota(jnp.int32, (zf.shape[0], STRIP), 1)
    acc_v = None
    acc_x = None
    acc_i = None
    for c in range(2):
        mv = None
        gv = None
        for g in range(CHUNK // STRIP):
            j0 = c * CHUNK + g * STRIP
            d_s = (sz + se[:, j0:j0 + STRIP]) - mm2[:, j0:j0 + STRIP]
            if mv is None:
                mv, gv = d_s, jnp.zeros_like(lane)
            else:
                upd = d_s < mv
                mv = jnp.where(upd, d_s, mv)
                gv = jnp.where(upd, g, gv)
        jv = gv * STRIP + lane + c * CHUNK
        m_c = jnp.min(mv, axis=1)
        i_c = jnp.min(jnp.where(mv == m_c[:, None], jv, N_E), axis=1)
        if acc_v is None:
            acc_v, acc_x, acc_i = m_c, m_c, i_c
        else:
            upd = m_c < acc_v
            acc_v = jnp.where(upd, m_c, acc_v)
            acc_x = jnp.where(upd, m_c, acc_x)
            acc_i = jnp.where(upd, i_c, acc_i)
        acc_v = acc_v.astype(jnp.bfloat16).astype(jnp.float32)
    idx_ref[...] = acc_i[:, None]
    loss_ref[...] = jnp.sum(acc_x).reshape(1, 1, 1)


def _sc_gather(table_hbm, idx_hbm, out_hbm, idx_v, rows_v, sem):
    info = plsc.get_sparse_core_info()
    nw = info.num_cores * info.num_subcores
    b_per_w = idx_hbm.shape[0] // nw
    wid = lax.axis_index("s") * info.num_cores + lax.axis_index("c")
    base = wid * b_per_w
    pltpu.sync_copy(idx_hbm.at[pl.ds(base, b_per_w)], idx_v)
    pltpu.async_copy(table_hbm.at[idx_v], rows_v, sem).wait()
    pltpu.sync_copy(rows_v, out_hbm.at[pl.ds(base, b_per_w)])


@functools.partial(jax.jit, static_argnames=())
def kernel(z, embedding_weight):
    B, C, H, W = z.shape
    n = B * H * W
    grid = n // TILE
    zp = jnp.transpose(z, (0, 2, 3, 1))
    zf = zp.reshape(n, E_DIM)
    sz = jnp.sum(zp ** 2, axis=3).reshape(n, 1)
    et = embedding_weight.T

    idx_col, loss_parts = pl.pallas_call(
        _vq_tile,
        grid=(grid,),
        in_specs=[
            pl.BlockSpec((TILE, E_DIM), lambda i: (i, 0)),
            pl.BlockSpec((TILE, 1), lambda i: (i, 0)),
            pl.BlockSpec((E_DIM, N_E), lambda i: (0, 0)),
        ],
        out_specs=[
            pl.BlockSpec((TILE, 1), lambda i: (i, 0)),
            pl.BlockSpec((1, 1, 1), lambda i: (i, 0, 0)),
        ],
        out_shape=[
            jax.ShapeDtypeStruct((n, 1), jnp.int32),
            jax.ShapeDtypeStruct((grid, 1, 1), jnp.float32),
        ],
        scratch_shapes=[pltpu.VMEM((1, N_E), jnp.float32)],
    )(zf, sz, et)

    idx_flat = idx_col.reshape(n)

    table = jnp.pad(embedding_weight, ((0, 0), (0, 128 - E_DIM)))

    info = plsc.get_sparse_core_info()
    nw = info.num_cores * info.num_subcores
    b_per_w = n // nw
    mesh = plsc.VectorSubcoreMesh(core_axis_name="c", subcore_axis_name="s")
    zq_pad = pl.kernel(
        _sc_gather,
        mesh=mesh,
        out_type=jax.ShapeDtypeStruct((n, 128), jnp.float32),
        scratch_types=[
            pltpu.VMEM((b_per_w,), jnp.int32),
            pltpu.VMEM((b_per_w, 128), jnp.float32),
            pltpu.SemaphoreType.DMA,
        ],
    )(table, idx_flat)
    zq_flat = zq_pad[:, :E_DIM]

    mse = jnp.sum(loss_parts) / (n * E_DIM)
    loss = mse + BETA * mse
    z_q_out = jnp.transpose(zq_flat.reshape(B, H, W, C), (0, 3, 1, 2))
    idx_out = idx_col.reshape(B, H, W)
    return (z_q_out, loss, idx_out)
